# Initial kernel scaffold; baseline (speedup 1.0000x reference)
import jax
import jax.numpy as jnp
from jax.experimental import pallas as pl


def kernel(exp_w, exp_b, exp_bn_gamma, exp_bn_beta, exp_bn_mean, exp_bn_var, dw_w, dw_b, dw_bn_gamma, dw_bn_beta, dw_bn_mean, dw_bn_var, point_w, point_b, point_bn_gamma, point_bn_beta, point_bn_mean, point_bn_var, shortcut_w, shortcut_b, shortcut_bn_gamma, shortcut_bn_beta, shortcut_bn_mean, shortcut_bn_var, se_w1, se_b1, se_w2, se_b2, x):
    raise NotImplementedError("write your pallas kernel here")


# speedup vs baseline: 2.4627x; 2.4627x over previous
import functools

import jax
import jax.numpy as jnp
from jax import lax
from jax.experimental import pallas as pl
from jax.experimental.pallas import tpu as pltpu

_EPS = 1e-5


def _hswish(v):
    return v * jnp.clip(v + 3.0, 0.0, 6.0) * (1.0 / 6.0)


def _round_up(v, m):
    return (v + m - 1) // m * m


def _fused_kernel(x_ref, we_ref, be_ref, wd_ref, bd_ref, w1_ref, b1_ref,
                  w2_ref, b2_ref, wp_ref, bp_ref, ws_ref, bs_ref, o_ref,
                  p0_ref, p1_ref, p2_ref, p3_ref, p4_ref, y_ref,
                  *, K, H, W, Wp, base, strip):
    R = H * Wp
    Cmid = we_ref.shape[1]
    p = K // 2
    planes = [p0_ref, p1_ref, p2_ref, p3_ref, p4_ref]
    xb = x_ref[0]

    mid = lax.dot_general(xb, we_ref[...], (((0,), (0,)), ((), ())),
                          preferred_element_type=jnp.float32)
    colv = lax.broadcasted_iota(jnp.int32, (R, 1), 0) % Wp
    colmask = (colv >= p) & (colv < p + W)
    mid = jnp.where(colmask, _hswish(mid + be_ref[...]), 0.0)

    P = p0_ref.shape[0]
    midb = mid.astype(jnp.bfloat16)
    for b in range(K):
        d = base - (b - p)
        planes[b][0:d, :] = jnp.zeros((d, Cmid), jnp.bfloat16)
        planes[b][d + R:P, :] = jnp.zeros((P - d - R, Cmid), jnp.bfloat16)
        planes[b][d:d + R, :] = midb

    wrow = [wd_ref[t:t + 1, :] for t in range(K * K)]
    bd = bd_ref[...]
    smask = colmask[0:strip]
    psum = jnp.zeros((1, Cmid), jnp.float32)
    ext = p * Wp
    for s0 in range(0, R, strip):
        acc = None
        for b in range(K):
            chunk = planes[b][s0 + base - ext:s0 + base + strip + ext, :]
            for a in range(K):
                tap = chunk[Wp * a:Wp * a + strip, :] * wrow[a * K + b]
                acc = tap if acc is None else acc + tap
        ys = _hswish(acc.astype(jnp.float32) + bd)
        psum = psum + jnp.sum(jnp.where(smask, ys, 0.0), axis=0, keepdims=True)
        y_ref[s0:s0 + strip, :] = ys

    pooled = psum * (1.0 / (H * W))
    h1 = jnp.dot(pooled, w1_ref[...], preferred_element_type=jnp.float32)
    h1 = jnp.clip(h1 + b1_ref[...], 0.0, 6.0)
    sc = jnp.dot(h1, w2_ref[...], preferred_element_type=jnp.float32)
    scale = _hswish(sc + b2_ref[...])

    gated = y_ref[...] * scale
    pt = lax.dot_general(wp_ref[...], gated, (((0,), (1,)), ((), ())),
                         preferred_element_type=jnp.float32)
    sct = lax.dot_general(ws_ref[...], xb, (((0,), (0,)), ((), ())),
                          preferred_element_type=jnp.float32)
    bp_col = jnp.transpose(bp_ref[...])
    bs_col = jnp.transpose(bs_ref[...])
    o_ref[0] = _hswish(pt + bp_col) + sct + bs_col


def _fold_pw(w, b, gamma, beta, mean, var):
    g = gamma * lax.rsqrt(var + _EPS)
    return w * g[None, :], ((b - mean) * g + beta).reshape(1, -1)


def kernel(exp_w, exp_b, exp_bn_gamma, exp_bn_beta, exp_bn_mean, exp_bn_var,
           dw_w, dw_b, dw_bn_gamma, dw_bn_beta, dw_bn_mean, dw_bn_var,
           point_w, point_b, point_bn_gamma, point_bn_beta, point_bn_mean,
           point_bn_var, shortcut_w, shortcut_b, shortcut_bn_gamma,
           shortcut_bn_beta, shortcut_bn_mean, shortcut_bn_var,
           se_w1, se_b1, se_w2, se_b2, x):
    N, Cin, H, W = x.shape
    K = dw_w.shape[0]
    Cmid = exp_w.shape[1]
    Cout = point_w.shape[1]
    Cse = se_w1.shape[1]

    we, be = _fold_pw(exp_w, exp_b, exp_bn_gamma, exp_bn_beta, exp_bn_mean,
                      exp_bn_var)
    wp, bp = _fold_pw(point_w, point_b, point_bn_gamma, point_bn_beta,
                      point_bn_mean, point_bn_var)
    ws, bs = _fold_pw(shortcut_w, shortcut_b, shortcut_bn_gamma,
                      shortcut_bn_beta, shortcut_bn_mean, shortcut_bn_var)
    gd = dw_bn_gamma * lax.rsqrt(dw_bn_var + _EPS)
    wd = (dw_w * gd[None, None, :]).reshape(K * K, Cmid).astype(jnp.bfloat16)
    bd = ((dw_b - dw_bn_mean) * gd + dw_bn_beta).reshape(1, Cmid)

    p = K // 2
    Wp = W + 2 * p
    R = H * Wp
    base = _round_up(p * Wp + p, 16)
    P = _round_up(base + R + p * Wp + p, 16)
    strip = 4 * Wp if (R % (4 * Wp) == 0) else Wp

    xp = jnp.pad(x, ((0, 0), (0, 0), (0, 0), (p, p))).reshape(N, Cin, R)

    out = pl.pallas_call(
        functools.partial(_fused_kernel, K=K, H=H, W=W, Wp=Wp, base=base,
                          strip=strip),
        out_shape=jax.ShapeDtypeStruct((N, Cout, R), jnp.float32),
        grid_spec=pltpu.PrefetchScalarGridSpec(
            num_scalar_prefetch=0,
            grid=(N,),
            in_specs=[
                pl.BlockSpec((1, Cin, R), lambda n: (n, 0, 0)),
                pl.BlockSpec((Cin, Cmid), lambda n: (0, 0)),
                pl.BlockSpec((1, Cmid), lambda n: (0, 0)),
                pl.BlockSpec((K * K, Cmid), lambda n: (0, 0)),
                pl.BlockSpec((1, Cmid), lambda n: (0, 0)),
                pl.BlockSpec((Cmid, Cse), lambda n: (0, 0)),
                pl.BlockSpec((1, Cse), lambda n: (0, 0)),
                pl.BlockSpec((Cse, Cmid), lambda n: (0, 0)),
                pl.BlockSpec((1, Cmid), lambda n: (0, 0)),
                pl.BlockSpec((Cmid, Cout), lambda n: (0, 0)),
                pl.BlockSpec((1, Cout), lambda n: (0, 0)),
                pl.BlockSpec((Cin, Cout), lambda n: (0, 0)),
                pl.BlockSpec((1, Cout), lambda n: (0, 0)),
            ],
            out_specs=pl.BlockSpec((1, Cout, R), lambda n: (n, 0, 0)),
            scratch_shapes=(
                [pltpu.VMEM((P, Cmid), jnp.bfloat16) for _ in range(K)]
                + [pltpu.VMEM((R, Cmid), jnp.float32)]),
        ),
        compiler_params=pltpu.CompilerParams(
            dimension_semantics=("parallel",)),
    )(xp, we, be, wd, bd, se_w1, se_b1.reshape(1, Cse), se_w2,
      se_b2.reshape(1, Cmid), wp, bp, ws, bs)

    return out.reshape(N, Cout, H, Wp)[:, :, :, p:p + W]
